# Initial kernel scaffold; baseline (speedup 1.0000x reference)
import jax
import jax.numpy as jnp
from jax.experimental import pallas as pl


def kernel(z, mask, embedding):
    raise NotImplementedError("write your pallas kernel here")


# speedup vs baseline: 2.8175x; 2.8175x over previous
import jax
import jax.numpy as jnp
from jax import lax
from jax.experimental import pallas as pl
from jax.experimental.pallas import tpu as pltpu

N_E = 1024
E_DIM = 256
BETA = 0.25
BT = 512
N_TOK = 16 * 1024
NB = N_TOK // BT


def _vq_body(z_ref, mask_ref, emb_ref,
             zq_ref, idx_ref, loss_ref, perp_ref,
             hist_acc, loss_acc):
    i = pl.program_id(0)
    zb = z_ref[0]
    emb = emb_ref[...]

    @pl.when(i == 0)
    def _init():
        hist_acc[...] = jnp.zeros((1, N_E), jnp.float32)
        loss_acc[0, 0] = 0.0

    zsq = jnp.sum(zb * zb, axis=1, keepdims=True)
    esq = lax.dot_general(
        jnp.ones((8, E_DIM), jnp.float32), emb * emb,
        dimension_numbers=(((1,), (1,)), ((), ())),
        preferred_element_type=jnp.float32,
        precision=lax.Precision.HIGHEST)[0:1]
    mm = lax.dot_general(
        zb, emb,
        dimension_numbers=(((1,), (1,)), ((), ())),
        preferred_element_type=jnp.float32)
    d = (zsq + esq) - 2.0 * mm

    dmin = jnp.min(d, axis=1, keepdims=True)
    jidx = lax.broadcasted_iota(jnp.int32, (BT, N_E), 1)
    idx = jnp.min(jnp.where(d == dmin, jidx, N_E), axis=1)
    idx_ref[0, 0, :] = idx

    one_hot = (jidx == idx[:, None]).astype(jnp.float32)
    zq = lax.dot_general(
        one_hot, emb,
        dimension_numbers=(((1,), (0,)), ((), ())),
        preferred_element_type=jnp.float32)
    zq_ref[0] = zb + (zq - zb)

    mb = mask_ref[0, 0, :]
    hist_acc[...] += jnp.sum(one_hot, axis=0, keepdims=True)
    loss_acc[0, 0] += jnp.sum(mb * dmin[:, 0])

    @pl.when(i == NB - 1)
    def _final():
        loss_ref[0, 0] = (1.0 + BETA) * loss_acc[0, 0] / (N_TOK * E_DIM)
        e_mean = hist_acc[...] * (1.0 / N_TOK)
        ent = jnp.sum(e_mean * jnp.log(e_mean + 1e-10))
        perp_ref[0, 0] = jnp.exp(-ent)


def kernel(z, mask, embedding):
    z3 = z.reshape(NB, BT, E_DIM)
    mask3 = mask.reshape(NB, 1, BT)
    zq, idx, loss, perp = pl.pallas_call(
        _vq_body,
        grid=(NB,),
        in_specs=[
            pl.BlockSpec((1, BT, E_DIM), lambda i: (i, 0, 0)),
            pl.BlockSpec((1, 1, BT), lambda i: (i, 0, 0)),
            pl.BlockSpec((N_E, E_DIM), lambda i: (0, 0)),
        ],
        out_specs=[
            pl.BlockSpec((1, BT, E_DIM), lambda i: (i, 0, 0)),
            pl.BlockSpec((1, 1, BT), lambda i: (i, 0, 0)),
            pl.BlockSpec(memory_space=pltpu.SMEM),
            pl.BlockSpec(memory_space=pltpu.SMEM),
        ],
        out_shape=[
            jax.ShapeDtypeStruct((NB, BT, E_DIM), jnp.float32),
            jax.ShapeDtypeStruct((NB, 1, BT), jnp.int32),
            jax.ShapeDtypeStruct((1, 1), jnp.float32),
            jax.ShapeDtypeStruct((1, 1), jnp.float32),
        ],
        scratch_shapes=[
            pltpu.VMEM((1, N_E), jnp.float32),
            pltpu.SMEM((1, 1), jnp.float32),
        ],
    )(z3, mask3, embedding)
    return (zq.reshape(z.shape), idx.reshape(N_TOK, 1),
            loss[0, 0], perp[0, 0])
